# Initial kernel scaffold; baseline (speedup 1.0000x reference)
import jax
import jax.numpy as jnp
from jax.experimental import pallas as pl


def kernel(scores, coords, features, batch_ids, offset):
    raise NotImplementedError("write your pallas kernel here")


# speedup vs baseline: 14.2890x; 14.2890x over previous
import functools

import jax
import jax.numpy as jnp
from jax.experimental import pallas as pl

K = 16
TAU = 0.1


def _gmax_body(f_ref, o_ref):
    o_ref[0, ...] = jnp.max(f_ref[...], axis=0, keepdims=True)


def _main_body(q_ref, kt_ref, sq_ref, sk_ref, f_ref, g_ref, o_ref,
               *, nb, n, r_blk):
    b = pl.program_id(0)
    r = pl.program_id(1)

    @pl.when(jnp.logical_and(b == 0, r == 0))
    def _init():
        o_ref[...] = jnp.zeros_like(o_ref)

    q = q_ref[...]
    d = None
    for c in range(3):
        diff = q[:, c:c + 1] - kt_ref[c:c + 1, :]
        d = diff * diff if d is None else d + diff * diff

    inf = jnp.float32(jnp.inf)
    dm = d
    for _ in range(K):
        m = jnp.min(dm, axis=1, keepdims=True)
        dm = jnp.where(dm <= m, inf, dm)
    sel = dm == inf

    sk = sk_ref[...]
    cnt = jnp.sum(jnp.where(sel, 1.0, 0.0), axis=1, keepdims=True)
    ssum = jnp.sum(jnp.where(sel, sk, 0.0), axis=1, keepdims=True)
    s2sum = jnp.sum(jnp.where(sel, sk * sk, 0.0), axis=1, keepdims=True)
    si = sq_ref[...]
    loc_part = jnp.sum(cnt * si * si - 2.0 * si * ssum + s2sum)

    f = f_ref[...]
    g = g_ref[0, ...]
    t = jnp.mean(2.0 * jax.nn.sigmoid((f - g) / TAU), axis=1, keepdims=True)
    glob_part = jnp.sum((si - t) ** 2)

    o_ref[...] += (loc_part / (n * K) + glob_part / n).reshape(1, 1)


def kernel(scores, coords, features, batch_ids, offset):
    n, c = features.shape
    b = offset.shape[0]
    nb = n // b
    r_blk = 128
    nblk = nb // r_blk

    gmax = pl.pallas_call(
        _gmax_body,
        grid=(b,),
        in_specs=[pl.BlockSpec((nb, c), lambda i: (i, 0))],
        out_specs=pl.BlockSpec((1, 1, c), lambda i: (i, 0, 0)),
        out_shape=jax.ShapeDtypeStruct((b, 1, c), jnp.float32),
    )(features)

    coords_t = coords.T
    scores_col = scores.reshape(n, 1)
    scores_row = scores.reshape(1, n)

    out = pl.pallas_call(
        functools.partial(_main_body, nb=nb, n=n, r_blk=r_blk),
        grid=(b, nblk),
        in_specs=[
            pl.BlockSpec((r_blk, 3), lambda i, j: (i * nblk + j, 0)),
            pl.BlockSpec((3, nb), lambda i, j: (0, i)),
            pl.BlockSpec((r_blk, 1), lambda i, j: (i * nblk + j, 0)),
            pl.BlockSpec((1, nb), lambda i, j: (0, i)),
            pl.BlockSpec((r_blk, c), lambda i, j: (i * nblk + j, 0)),
            pl.BlockSpec((1, 1, c), lambda i, j: (i, 0, 0)),
        ],
        out_specs=pl.BlockSpec((1, 1), lambda i, j: (0, 0)),
        out_shape=jax.ShapeDtypeStruct((1, 1), jnp.float32),
    )(coords, coords_t, scores_col, scores_row, features, gmax)

    return out[0, 0]


# speedup vs baseline: 16.0466x; 1.1230x over previous
import functools

import jax
import jax.numpy as jnp
from jax.experimental import pallas as pl

K = 16
TAU = 0.1


def _gmax_body(f_ref, o_ref):
    o_ref[0, ...] = jnp.max(f_ref[...], axis=0, keepdims=True)


def _main_body(q_ref, kt_ref, sq_ref, w_ref, f_ref, g_ref, o_ref,
               *, nb, n, r_blk):
    b = pl.program_id(0)
    r = pl.program_id(1)

    @pl.when(jnp.logical_and(b == 0, r == 0))
    def _init():
        o_ref[...] = jnp.zeros_like(o_ref)

    q = q_ref[...]
    kt = kt_ref[...]
    qn = jnp.sum(q * q, axis=1, keepdims=True)
    kn = jnp.sum(kt * kt, axis=0, keepdims=True)
    d = jnp.dot(q * -2.0, kt, preferred_element_type=jnp.float32)
    d = (d + qn) + kn

    inf = jnp.float32(jnp.inf)
    dm = d
    for _ in range(K):
        m = jnp.min(dm, axis=1, keepdims=True)
        dm = jnp.where(dm <= m, inf, dm)
    sel = jnp.where(dm == inf, 1.0, 0.0)

    moms = jnp.dot(sel, w_ref[...], preferred_element_type=jnp.float32)
    cnt = moms[:, 0:1]
    ssum = moms[:, 1:2]
    s2sum = moms[:, 2:3]
    si = sq_ref[...]
    loc_part = jnp.sum(cnt * si * si - 2.0 * si * ssum + s2sum)

    f = f_ref[...]
    g = g_ref[0, ...]
    t = jnp.mean(2.0 * jax.nn.sigmoid((f - g) / TAU), axis=1, keepdims=True)
    glob_part = jnp.sum((si - t) ** 2)

    o_ref[...] += (loc_part / (n * K) + glob_part / n).reshape(1, 1)


def kernel(scores, coords, features, batch_ids, offset):
    n, c = features.shape
    b = offset.shape[0]
    nb = n // b
    r_blk = 128
    nblk = nb // r_blk

    gmax = pl.pallas_call(
        _gmax_body,
        grid=(b,),
        in_specs=[pl.BlockSpec((nb, c), lambda i: (i, 0))],
        out_specs=pl.BlockSpec((1, 1, c), lambda i: (i, 0, 0)),
        out_shape=jax.ShapeDtypeStruct((b, 1, c), jnp.float32),
    )(features)

    coords_t = coords.T
    scores_col = scores.reshape(n, 1)
    w = jnp.stack(
        [jnp.ones_like(scores), scores, scores * scores, scores * 0.0],
        axis=1)

    out = pl.pallas_call(
        functools.partial(_main_body, nb=nb, n=n, r_blk=r_blk),
        grid=(b, nblk),
        in_specs=[
            pl.BlockSpec((r_blk, 3), lambda i, j: (i * nblk + j, 0)),
            pl.BlockSpec((3, nb), lambda i, j: (0, i)),
            pl.BlockSpec((r_blk, 1), lambda i, j: (i * nblk + j, 0)),
            pl.BlockSpec((nb, 4), lambda i, j: (i, 0)),
            pl.BlockSpec((r_blk, c), lambda i, j: (i * nblk + j, 0)),
            pl.BlockSpec((1, 1, c), lambda i, j: (i, 0, 0)),
        ],
        out_specs=pl.BlockSpec((1, 1), lambda i, j: (0, 0)),
        out_shape=jax.ShapeDtypeStruct((1, 1), jnp.float32),
    )(coords, coords_t, scores_col, w, features, gmax)

    return out[0, 0]


# speedup vs baseline: 25.1418x; 1.5668x over previous
import functools

import jax
import jax.numpy as jnp
from jax.experimental import pallas as pl

K = 16
TAU = 0.1


def _gmax_body(f_ref, o_ref):
    o_ref[0, ...] = jnp.max(f_ref[...], axis=0, keepdims=True)


def _main_body(q_ref, kt_ref, sq_ref, w_ref, f_ref, g_ref, o_ref,
               *, nb, n, r_blk):
    b = pl.program_id(0)
    r = pl.program_id(1)

    @pl.when(jnp.logical_and(b == 0, r == 0))
    def _init():
        o_ref[...] = jnp.zeros_like(o_ref)

    q = q_ref[...]
    kt = kt_ref[...]
    qn = jnp.sum(q * q, axis=1, keepdims=True)
    kn = jnp.sum(kt * kt, axis=0, keepdims=True)
    d = jnp.dot(q * -2.0, kt, preferred_element_type=jnp.float32)
    d = (d + qn) + kn

    inf = jnp.bfloat16(jnp.inf)
    dm = d.astype(jnp.bfloat16)
    for _ in range(K):
        m = jnp.min(dm, axis=1, keepdims=True)
        dm = jnp.where(dm <= m, inf, dm)
    sel = jnp.where(dm == inf, jnp.bfloat16(1), jnp.bfloat16(0))

    moms = jnp.dot(sel, w_ref[...], preferred_element_type=jnp.float32)
    cnt = moms[:, 0:1]
    ssum = moms[:, 1:2]
    s2sum = moms[:, 2:3]
    si = sq_ref[...]
    loc_part = jnp.sum(
        (cnt * si * si - 2.0 * si * ssum + s2sum) * (K / cnt))

    f = f_ref[...]
    g = g_ref[0, ...]
    t = jnp.mean(2.0 * jax.nn.sigmoid((f - g) / TAU), axis=1, keepdims=True)
    glob_part = jnp.sum((si - t) ** 2)

    o_ref[...] += (loc_part / (n * K) + glob_part / n).reshape(1, 1)


def kernel(scores, coords, features, batch_ids, offset):
    n, c = features.shape
    b = offset.shape[0]
    nb = n // b
    r_blk = 128
    nblk = nb // r_blk

    gmax = pl.pallas_call(
        _gmax_body,
        grid=(b,),
        in_specs=[pl.BlockSpec((nb, c), lambda i: (i, 0))],
        out_specs=pl.BlockSpec((1, 1, c), lambda i: (i, 0, 0)),
        out_shape=jax.ShapeDtypeStruct((b, 1, c), jnp.float32),
    )(features)

    coords_t = coords.T
    scores_col = scores.reshape(n, 1)
    w = jnp.stack(
        [jnp.ones_like(scores), scores, scores * scores, scores * 0.0],
        axis=1).astype(jnp.bfloat16)

    out = pl.pallas_call(
        functools.partial(_main_body, nb=nb, n=n, r_blk=r_blk),
        grid=(b, nblk),
        in_specs=[
            pl.BlockSpec((r_blk, 3), lambda i, j: (i * nblk + j, 0)),
            pl.BlockSpec((3, nb), lambda i, j: (0, i)),
            pl.BlockSpec((r_blk, 1), lambda i, j: (i * nblk + j, 0)),
            pl.BlockSpec((nb, 4), lambda i, j: (i, 0)),
            pl.BlockSpec((r_blk, c), lambda i, j: (i * nblk + j, 0)),
            pl.BlockSpec((1, 1, c), lambda i, j: (i, 0, 0)),
        ],
        out_specs=pl.BlockSpec((1, 1), lambda i, j: (0, 0)),
        out_shape=jax.ShapeDtypeStruct((1, 1), jnp.float32),
    )(coords, coords_t, scores_col, w, features, gmax)

    return out[0, 0]


# speedup vs baseline: 45.2833x; 1.8011x over previous
import functools

import jax
import jax.numpy as jnp
from jax.experimental import pallas as pl

K = 16
TAU = 0.1
FOLD = 16


def _gmax_body(f_ref, o_ref):
    o_ref[0, ...] = jnp.max(f_ref[...], axis=0, keepdims=True)


def _main_body(qa_ref, ka_ref, sq_ref, w_ref, f_ref, g_ref, o_ref,
               *, nb, n, r_blk):
    b = pl.program_id(0)
    r = pl.program_id(1)

    @pl.when(jnp.logical_and(b == 0, r == 0))
    def _init():
        o_ref[...] = jnp.zeros_like(o_ref)

    d = jnp.dot(qa_ref[...], ka_ref[...], preferred_element_type=jnp.float32)
    dm = d.astype(jnp.bfloat16)

    w = nb // FOLD
    fm = dm[:, :w]
    for t in range(1, FOLD):
        fm = jnp.minimum(fm, dm[:, t * w:(t + 1) * w])

    inf = jnp.bfloat16(jnp.inf)
    for _ in range(K - 1):
        m = jnp.min(fm, axis=1, keepdims=True)
        fm = jnp.where(fm <= m, inf, fm)
    t_thr = jnp.min(fm, axis=1, keepdims=True)

    sel = jnp.where(dm <= t_thr, jnp.bfloat16(1), jnp.bfloat16(0))

    moms = jnp.dot(sel, w_ref[...], preferred_element_type=jnp.float32)
    si = sq_ref[...]
    cnt = moms[:, 0:1] - 1.0
    ssum = moms[:, 1:2] - si
    s2sum = moms[:, 2:3] - si * si
    loc_part = jnp.sum(
        (cnt * si * si - 2.0 * si * ssum + s2sum) * ((K - 1.0) / cnt))

    f = f_ref[...]
    g = g_ref[0, ...]
    t = jnp.mean(2.0 * jax.nn.sigmoid((f - g) / TAU), axis=1, keepdims=True)
    glob_part = jnp.sum((si - t) ** 2)

    o_ref[...] += (loc_part / (n * K) + glob_part / n).reshape(1, 1)


def kernel(scores, coords, features, batch_ids, offset):
    n, c = features.shape
    b = offset.shape[0]
    nb = n // b
    r_blk = 128
    nblk = nb // r_blk

    gmax = pl.pallas_call(
        _gmax_body,
        grid=(b,),
        in_specs=[pl.BlockSpec((nb, c), lambda i: (i, 0))],
        out_specs=pl.BlockSpec((1, 1, c), lambda i: (i, 0, 0)),
        out_shape=jax.ShapeDtypeStruct((b, 1, c), jnp.float32),
    )(features)

    kn = jnp.sum(coords * coords, axis=1, keepdims=True)
    one = jnp.ones_like(kn)
    qa = jnp.concatenate([coords * -2.0, one, kn], axis=1)
    ka = jnp.concatenate([coords, kn, one], axis=1).T
    scores_col = scores.reshape(n, 1)
    w = jnp.stack(
        [jnp.ones_like(scores), scores, scores * scores, scores * 0.0],
        axis=1).astype(jnp.bfloat16)

    out = pl.pallas_call(
        functools.partial(_main_body, nb=nb, n=n, r_blk=r_blk),
        grid=(b, nblk),
        in_specs=[
            pl.BlockSpec((r_blk, 5), lambda i, j: (i * nblk + j, 0)),
            pl.BlockSpec((5, nb), lambda i, j: (0, i)),
            pl.BlockSpec((r_blk, 1), lambda i, j: (i * nblk + j, 0)),
            pl.BlockSpec((nb, 4), lambda i, j: (i, 0)),
            pl.BlockSpec((r_blk, c), lambda i, j: (i * nblk + j, 0)),
            pl.BlockSpec((1, 1, c), lambda i, j: (i, 0, 0)),
        ],
        out_specs=pl.BlockSpec((1, 1), lambda i, j: (0, 0)),
        out_shape=jax.ShapeDtypeStruct((1, 1), jnp.float32),
    )(qa, ka, scores_col, w, features, gmax)

    return out[0, 0]


# speedup vs baseline: 63.6047x; 1.4046x over previous
import functools

import jax
import jax.numpy as jnp
from jax.experimental import pallas as pl

K = 16
TAU = 0.1
FOLD = 16


def _gmax_body(f_ref, o_ref):
    o_ref[0, ...] = jnp.max(f_ref[...], axis=0, keepdims=True)


def _main_body(qa_ref, ka_ref, sq_ref, w_ref, f_ref, g_ref, o_ref,
               *, nb, n, r_blk):
    b = pl.program_id(0)
    r = pl.program_id(1)

    @pl.when(jnp.logical_and(b == 0, r == 0))
    def _init():
        o_ref[...] = jnp.zeros_like(o_ref)

    d = jnp.dot(qa_ref[...], ka_ref[...], preferred_element_type=jnp.float32)
    dm = d.astype(jnp.bfloat16)

    w = nb // FOLD
    fm = dm[:, :w]
    for t in range(1, FOLD):
        fm = jnp.minimum(fm, dm[:, t * w:(t + 1) * w])

    inf = jnp.bfloat16(jnp.inf)
    for _ in range(K - 1):
        m = jnp.min(fm, axis=1, keepdims=True)
        fm = jnp.where(fm <= m, inf, fm)
    t_thr = jnp.min(fm, axis=1, keepdims=True)

    sel = jnp.where(dm <= t_thr, jnp.bfloat16(1), jnp.bfloat16(0))

    moms = jnp.dot(sel, w_ref[...], preferred_element_type=jnp.float32)
    si = sq_ref[...]
    cnt = moms[:, 0:1] - 1.0
    ssum = moms[:, 1:2] - si
    s2sum = moms[:, 2:3] - si * si
    loc_part = jnp.sum(
        (cnt * si * si - 2.0 * si * ssum + s2sum) * ((K - 1.0) / cnt))

    f = f_ref[...]
    g = g_ref[0, ...]
    t = jnp.mean(2.0 * jax.nn.sigmoid((f - g) / TAU), axis=1, keepdims=True)
    glob_part = jnp.sum((si - t) ** 2)

    o_ref[...] += (loc_part / (n * K) + glob_part / n).reshape(1, 1)


def kernel(scores, coords, features, batch_ids, offset):
    n, c = features.shape
    b = offset.shape[0]
    nb = n // b
    r_blk = 256
    nblk = nb // r_blk

    gmax = pl.pallas_call(
        _gmax_body,
        grid=(b,),
        in_specs=[pl.BlockSpec((nb, c), lambda i: (i, 0))],
        out_specs=pl.BlockSpec((1, 1, c), lambda i: (i, 0, 0)),
        out_shape=jax.ShapeDtypeStruct((b, 1, c), jnp.float32),
    )(features)

    kn = jnp.sum(coords * coords, axis=1, keepdims=True)
    one = jnp.ones_like(kn)
    qa = jnp.concatenate([coords * -2.0, one, kn], axis=1)
    ka = jnp.concatenate([coords, kn, one], axis=1).T
    scores_col = scores.reshape(n, 1)
    w = jnp.stack(
        [jnp.ones_like(scores), scores, scores * scores, scores * 0.0],
        axis=1).astype(jnp.bfloat16)

    out = pl.pallas_call(
        functools.partial(_main_body, nb=nb, n=n, r_blk=r_blk),
        grid=(b, nblk),
        in_specs=[
            pl.BlockSpec((r_blk, 5), lambda i, j: (i * nblk + j, 0)),
            pl.BlockSpec((5, nb), lambda i, j: (0, i)),
            pl.BlockSpec((r_blk, 1), lambda i, j: (i * nblk + j, 0)),
            pl.BlockSpec((nb, 4), lambda i, j: (i, 0)),
            pl.BlockSpec((r_blk, c), lambda i, j: (i * nblk + j, 0)),
            pl.BlockSpec((1, 1, c), lambda i, j: (i, 0, 0)),
        ],
        out_specs=pl.BlockSpec((1, 1), lambda i, j: (0, 0)),
        out_shape=jax.ShapeDtypeStruct((1, 1), jnp.float32),
    )(qa, ka, scores_col, w, features, gmax)

    return out[0, 0]
